# Initial kernel scaffold; baseline (speedup 1.0000x reference)
import jax
import jax.numpy as jnp
from jax.experimental import pallas as pl


def kernel(x_sequence, W_s1, b_s1, W_d1, b_d1, W_s2, b_s2, W_d2, b_d2, W_ih, W_hh, b_ih, b_hh, W_p, b_p, edge_index_sequence):
    raise NotImplementedError("write your pallas kernel here")


# speedup vs baseline: 10.5435x; 10.5435x over previous
import functools

import jax
import jax.numpy as jnp
from jax import lax
from jax.experimental import pallas as pl
from jax.experimental.pallas import tpu as pltpu
from jax.experimental.pallas import tpu_sc as plsc

N = 10000
T = 8
F_DIM = 128
H = 128
E = 320000
ALPHA = 0.5

NC = 2
NS = 16
EPT = E // NS
CHB = 125
NCH = EPT // CHB
PAIRS = T * 2
PPC = PAIRS // NC
ROWS_PT = N // NS

_mesh = plsc.VectorSubcoreMesh(core_axis_name="c", subcore_axis_name="s")


def _deg_body(gidx_hbm, ones_hbm, zeros_hbm, deg_hbm, idx_v, ones_v, bounce_v,
              sem, hist_sp):
    c = lax.axis_index("c")
    s = lax.axis_index("s")
    half = N * PPC
    base = c * half + s * (half // NS)
    pltpu.sync_copy(ones_hbm, ones_v)
    pltpu.sync_copy(zeros_hbm, bounce_v)
    pltpu.sync_copy(bounce_v, hist_sp.at[pl.ds(base, half // NS)])
    plsc.subcore_barrier()
    for tt in range(T // NC):
        for dd in range(2):
            t = c * (T // NC) + tt
            pltpu.sync_copy(gidx_hbm.at[t, dd, s], idx_v)

            @pl.loop(0, NCH)
            def _(ch):
                pltpu.sync_copy(ones_v, hist_sp.at[idx_v.at[ch]], add=True)

    plsc.subcore_barrier()
    pltpu.sync_copy(hist_sp.at[pl.ds(base, half // NS)], bounce_v)
    pltpu.sync_copy(bounce_v, deg_hbm.at[pl.ds(base, half // NS)])


_deg_call = pl.kernel(
    _deg_body,
    out_type=jax.ShapeDtypeStruct((PAIRS * N,), jnp.float32),
    mesh=_mesh,
    compiler_params=pltpu.CompilerParams(use_tc_tiling_on_sc=False),
    scratch_types=[
        pltpu.VMEM((NCH, CHB), jnp.int32),
        pltpu.VMEM((CHB,), jnp.float32),
        pltpu.VMEM((N * PPC // NS,), jnp.float32),
        pltpu.SemaphoreType.DMA,
        pltpu.VMEM_SHARED((PAIRS * N,), jnp.float32),
    ],
)


def _agg_body(table_hbm, gidx_hbm, sidx_hbm, zrows_hbm, out_hbm,
              gidx_v, sidx_v, rows_v, sem, acc_sp):
    c = lax.axis_index("c")
    s = lax.axis_index("s")
    for tt in range(T // NC):
        for dd in range(2):
            t = c * (T // NC) + tt
            pltpu.sync_copy(zrows_hbm, acc_sp.at[pl.ds(s * ROWS_PT, ROWS_PT)])
            plsc.subcore_barrier()
            for hh in range(2):
                pltpu.sync_copy(gidx_hbm.at[t, dd, s, pl.ds(hh * (NCH // 2), NCH // 2)],
                                gidx_v)
                pltpu.sync_copy(sidx_hbm.at[t, dd, s, pl.ds(hh * (NCH // 2), NCH // 2)],
                                sidx_v)

                @pl.loop(0, NCH // 2)
                def _(ch):
                    pltpu.async_copy(table_hbm.at[gidx_v.at[ch]], rows_v, sem).wait()
                    pltpu.sync_copy(rows_v, acc_sp.at[sidx_v.at[ch]], add=True)

            plsc.subcore_barrier()
            pltpu.sync_copy(acc_sp.at[pl.ds(s * ROWS_PT, ROWS_PT)],
                            out_hbm.at[t, dd, pl.ds(s * ROWS_PT, ROWS_PT)])


_agg_call = pl.kernel(
    _agg_body,
    out_type=jax.ShapeDtypeStruct((T, 2, N, F_DIM), jnp.float32),
    mesh=_mesh,
    compiler_params=pltpu.CompilerParams(use_tc_tiling_on_sc=False),
    scratch_types=[
        pltpu.VMEM((NCH // 2, CHB), jnp.int32),
        pltpu.VMEM((NCH // 2, CHB), jnp.int32),
        pltpu.VMEM((CHB, F_DIM), jnp.float32),
        pltpu.SemaphoreType.DMA,
        pltpu.VMEM_SHARED((N, F_DIM), jnp.float32),
    ],
)


BN = 2000
NB = N // BN


def _inv_sqrt(d):
    return jnp.where(d > 0, lax.rsqrt(jnp.maximum(d, 1e-12)), 0.0)


def _scale_body(x_ref, di_ref, do_ref, out_ref):
    x = x_ref[0]
    inv_i = _inv_sqrt(di_ref[0])
    inv_o = _inv_sqrt(do_ref[0])
    out_ref[0, 0] = x * inv_i
    out_ref[0, 1] = x * inv_o


_scale_call = pl.pallas_call(
    _scale_body,
    grid=(T, NB),
    in_specs=[
        pl.BlockSpec((1, BN, F_DIM), lambda t, n: (t, n, 0)),
        pl.BlockSpec((1, BN, 1), lambda t, n: (t, n, 0)),
        pl.BlockSpec((1, BN, 1), lambda t, n: (t, n, 0)),
    ],
    out_specs=pl.BlockSpec((1, 2, BN, F_DIM), lambda t, n: (t, 0, n, 0)),
    out_shape=jax.ShapeDtypeStruct((T, 2, N, F_DIM), jnp.float32),
)


def _combine_body(u_ref, di_ref, do_ref, ws_ref, bs_ref, wd_ref, bd_ref,
                  out_ref, *, relu_and_scale):
    inv_i = _inv_sqrt(di_ref[0])
    inv_o = _inv_sqrt(do_ref[0])
    agg_fwd = u_ref[0, 0] * inv_o
    agg_rev = u_ref[0, 1] * inv_i
    dn = (((1,), (1,)), ((), ()))
    h = (ALPHA * (lax.dot_general(agg_fwd, ws_ref[...], dn,
                                  preferred_element_type=jnp.float32)
                  + bs_ref[...])
         + (1.0 - ALPHA) * (lax.dot_general(agg_rev, wd_ref[...], dn,
                                            preferred_element_type=jnp.float32)
                            + bd_ref[...]))
    if relu_and_scale:
        h = jnp.maximum(h, 0.0)
        out_ref[0, 0] = h * inv_i
        out_ref[0, 1] = h * inv_o
    else:
        out_ref[0] = h


def _make_combine(relu_and_scale):
    if relu_and_scale:
        out_specs = pl.BlockSpec((1, 2, BN, F_DIM), lambda t, n: (t, 0, n, 0))
        out_shape = jax.ShapeDtypeStruct((T, 2, N, F_DIM), jnp.float32)
    else:
        out_specs = pl.BlockSpec((1, BN, F_DIM), lambda t, n: (t, n, 0))
        out_shape = jax.ShapeDtypeStruct((T, N, F_DIM), jnp.float32)
    return pl.pallas_call(
        functools.partial(_combine_body, relu_and_scale=relu_and_scale),
        grid=(T, NB),
        in_specs=[
            pl.BlockSpec((1, 2, BN, F_DIM), lambda t, n: (t, 0, n, 0)),
            pl.BlockSpec((1, BN, 1), lambda t, n: (t, n, 0)),
            pl.BlockSpec((1, BN, 1), lambda t, n: (t, n, 0)),
            pl.BlockSpec((H, F_DIM), lambda t, n: (0, 0)),
            pl.BlockSpec((1, H), lambda t, n: (0, 0)),
            pl.BlockSpec((H, F_DIM), lambda t, n: (0, 0)),
            pl.BlockSpec((1, H), lambda t, n: (0, 0)),
        ],
        out_specs=out_specs,
        out_shape=out_shape,
    )


_combine1_call = _make_combine(True)
_combine2_call = _make_combine(False)

BL = 2000
NBL = N // BL


def _lstm_body(seq_ref, wih_ref, whh_ref, bih_ref, bhh_ref, wp_ref, bp_ref,
               out_ref):
    dn = (((1,), (1,)), ((), ()))
    b = bih_ref[...] + bhh_ref[...]
    h = jnp.zeros((BL, H), jnp.float32)
    c = jnp.zeros((BL, H), jnp.float32)
    for t in range(T):
        xt = seq_ref[t]
        gates = (lax.dot_general(xt, wih_ref[...], dn,
                                 preferred_element_type=jnp.float32)
                 + lax.dot_general(h, whh_ref[...], dn,
                                   preferred_element_type=jnp.float32)
                 + b)
        i = jax.nn.sigmoid(gates[:, 0:H])
        f = jax.nn.sigmoid(gates[:, H:2 * H])
        g = jnp.tanh(gates[:, 2 * H:3 * H])
        o = jax.nn.sigmoid(gates[:, 3 * H:4 * H])
        c = f * c + i * g
        h = o * jnp.tanh(c)
    out_ref[...] = (lax.dot_general(h, wp_ref[...], dn,
                                    preferred_element_type=jnp.float32)
                    + bp_ref[...])


_lstm_call = pl.pallas_call(
    _lstm_body,
    grid=(NBL,),
    in_specs=[
        pl.BlockSpec((T, BL, H), lambda n: (0, n, 0)),
        pl.BlockSpec((4 * H, H), lambda n: (0, 0)),
        pl.BlockSpec((4 * H, H), lambda n: (0, 0)),
        pl.BlockSpec((1, 4 * H), lambda n: (0, 0)),
        pl.BlockSpec((1, 4 * H), lambda n: (0, 0)),
        pl.BlockSpec((F_DIM, H), lambda n: (0, 0)),
        pl.BlockSpec((1, F_DIM), lambda n: (0, 0)),
    ],
    out_specs=pl.BlockSpec((BL, F_DIM), lambda n: (n, 0)),
    out_shape=jax.ShapeDtypeStruct((N, F_DIM), jnp.float32),
)


def kernel(x_sequence, W_s1, b_s1, W_d1, b_d1, W_s2, b_s2, W_d2, b_d2,
           W_ih, W_hh, b_ih, b_hh, W_p, b_p, edge_index_sequence):
    ei = edge_index_sequence.astype(jnp.int32)
    gidx = jnp.flip(ei, axis=1)
    offs = (jnp.arange(T, dtype=jnp.int32)[:, None] * 2
            + jnp.arange(2, dtype=jnp.int32)[None, :]) * N
    gidx = gidx + offs[:, :, None]
    gidx = gidx.reshape(T, 2, NS, NCH, CHB)
    sidx = ei.reshape(T, 2, NS, NCH, CHB)

    ones_chb = jnp.ones((CHB,), jnp.float32)
    zeros_hist = jnp.zeros((N * PPC // NS,), jnp.float32)
    zeros_rows = jnp.zeros((ROWS_PT, F_DIM), jnp.float32)

    deg = _deg_call(gidx, ones_chb, zeros_hist).reshape(T, 2, N)
    deg_in = deg[:, 0, :, None]
    deg_out = deg[:, 1, :, None]

    xcat = _scale_call(x_sequence, deg_in, deg_out)
    u1 = _agg_call(xcat.reshape(PAIRS * N, F_DIM), gidx, sidx, zeros_rows)
    h1cat = _combine1_call(u1, deg_in, deg_out, W_s1, b_s1.reshape(1, H),
                           W_d1, b_d1.reshape(1, H))
    u2 = _agg_call(h1cat.reshape(PAIRS * N, F_DIM), gidx, sidx, zeros_rows)
    h2 = _combine2_call(u2, deg_in, deg_out, W_s2, b_s2.reshape(1, H),
                        W_d2, b_d2.reshape(1, H))
    out = _lstm_call(h2, W_ih, W_hh, b_ih.reshape(1, 4 * H),
                     b_hh.reshape(1, 4 * H), W_p, b_p.reshape(1, F_DIM))
    return out


# speedup vs baseline: 13.1826x; 1.2503x over previous
import functools

import jax
import jax.numpy as jnp
from jax import lax
from jax.experimental import pallas as pl
from jax.experimental.pallas import tpu as pltpu
from jax.experimental.pallas import tpu_sc as plsc

N = 10000
T = 8
F_DIM = 128
H = 128
E = 320000
ALPHA = 0.5

NC = 2
NS = 16
EPT = E // NS
CHB = 125
NCH = EPT // CHB
PAIRS = T * 2
PPC = PAIRS // NC
ROWS_PT = N // NS

_mesh = plsc.VectorSubcoreMesh(core_axis_name="c", subcore_axis_name="s")


def _deg_body(gidx_hbm, ones_hbm, zeros_hbm, deg_hbm, idx_v, ones_v, bounce_v,
              sem, hist_sp):
    c = lax.axis_index("c")
    s = lax.axis_index("s")
    half = N * PPC
    base = c * half + s * (half // NS)
    pltpu.sync_copy(ones_hbm, ones_v)
    pltpu.sync_copy(zeros_hbm, bounce_v)
    pltpu.sync_copy(bounce_v, hist_sp.at[pl.ds(base, half // NS)])
    plsc.subcore_barrier()
    for tt in range(T // NC):
        for dd in range(2):
            t = c * (T // NC) + tt
            pltpu.sync_copy(gidx_hbm.at[t, dd, s], idx_v)

            @pl.loop(0, NCH)
            def _(ch):
                pltpu.sync_copy(ones_v, hist_sp.at[idx_v.at[ch]], add=True)

    plsc.subcore_barrier()
    pltpu.sync_copy(hist_sp.at[pl.ds(base, half // NS)], bounce_v)
    pltpu.sync_copy(bounce_v, deg_hbm.at[pl.ds(base, half // NS)])


_deg_call = pl.kernel(
    _deg_body,
    out_type=jax.ShapeDtypeStruct((PAIRS * N,), jnp.float32),
    mesh=_mesh,
    compiler_params=pltpu.CompilerParams(use_tc_tiling_on_sc=False),
    scratch_types=[
        pltpu.VMEM((NCH, CHB), jnp.int32),
        pltpu.VMEM((CHB,), jnp.float32),
        pltpu.VMEM((N * PPC // NS,), jnp.float32),
        pltpu.SemaphoreType.DMA,
        pltpu.VMEM_SHARED((PAIRS * N,), jnp.float32),
    ],
)


NCHQ = NCH // 4


def _agg_body(table_hbm, gidx_hbm, sidx_hbm, zrows_hbm, out_hbm,
              gidx_v, sidx_v, rows0_v, rows1_v, sem, acc_sp):
    c = lax.axis_index("c")
    s = lax.axis_index("s")

    def wait_rows(dst):
        pltpu.make_async_copy(table_hbm.at[pl.ds(0, CHB)], dst, sem).wait()

    for tt in range(T // NC):
        for dd in range(2):
            t = c * (T // NC) + tt
            pltpu.sync_copy(zrows_hbm, acc_sp.at[pl.ds(s * ROWS_PT, ROWS_PT)])
            plsc.subcore_barrier()
            for hh in range(4):
                pltpu.sync_copy(gidx_hbm.at[t, dd, s, pl.ds(hh * NCHQ, NCHQ)],
                                gidx_v)
                pltpu.sync_copy(sidx_hbm.at[t, dd, s, pl.ds(hh * NCHQ, NCHQ)],
                                sidx_v)
                pltpu.async_copy(table_hbm.at[gidx_v.at[0]], rows0_v, sem)

                @pl.loop(0, NCHQ // 2 - 1)
                def _(i):
                    ch = 2 * i
                    wait_rows(rows0_v)
                    pltpu.async_copy(table_hbm.at[gidx_v.at[ch + 1]], rows1_v, sem)
                    pltpu.sync_copy(rows0_v, acc_sp.at[sidx_v.at[ch]], add=True)
                    wait_rows(rows1_v)
                    pltpu.async_copy(table_hbm.at[gidx_v.at[ch + 2]], rows0_v, sem)
                    pltpu.sync_copy(rows1_v, acc_sp.at[sidx_v.at[ch + 1]], add=True)

                wait_rows(rows0_v)
                pltpu.async_copy(table_hbm.at[gidx_v.at[NCHQ - 1]], rows1_v, sem)
                pltpu.sync_copy(rows0_v, acc_sp.at[sidx_v.at[NCHQ - 2]], add=True)
                wait_rows(rows1_v)
                pltpu.sync_copy(rows1_v, acc_sp.at[sidx_v.at[NCHQ - 1]], add=True)

            plsc.subcore_barrier()
            pltpu.sync_copy(acc_sp.at[pl.ds(s * ROWS_PT, ROWS_PT)],
                            out_hbm.at[t, dd, pl.ds(s * ROWS_PT, ROWS_PT)])


_agg_call = pl.kernel(
    _agg_body,
    out_type=jax.ShapeDtypeStruct((T, 2, N, F_DIM), jnp.float32),
    mesh=_mesh,
    compiler_params=pltpu.CompilerParams(use_tc_tiling_on_sc=False),
    scratch_types=[
        pltpu.VMEM((NCHQ, CHB), jnp.int32),
        pltpu.VMEM((NCHQ, CHB), jnp.int32),
        pltpu.VMEM((CHB, F_DIM), jnp.float32),
        pltpu.VMEM((CHB, F_DIM), jnp.float32),
        pltpu.SemaphoreType.DMA,
        pltpu.VMEM_SHARED((N, F_DIM), jnp.float32),
    ],
)


BN = 2000
NB = N // BN


def _inv_sqrt(d):
    return jnp.where(d > 0, lax.rsqrt(jnp.maximum(d, 1e-12)), 0.0)


def _scale_body(x_ref, di_ref, do_ref, out_ref):
    x = x_ref[0]
    inv_i = _inv_sqrt(di_ref[0])
    inv_o = _inv_sqrt(do_ref[0])
    out_ref[0, 0] = x * inv_i
    out_ref[0, 1] = x * inv_o


_scale_call = pl.pallas_call(
    _scale_body,
    grid=(T, NB),
    in_specs=[
        pl.BlockSpec((1, BN, F_DIM), lambda t, n: (t, n, 0)),
        pl.BlockSpec((1, BN, 1), lambda t, n: (t, n, 0)),
        pl.BlockSpec((1, BN, 1), lambda t, n: (t, n, 0)),
    ],
    out_specs=pl.BlockSpec((1, 2, BN, F_DIM), lambda t, n: (t, 0, n, 0)),
    out_shape=jax.ShapeDtypeStruct((T, 2, N, F_DIM), jnp.float32),
)


def _combine_body(u_ref, di_ref, do_ref, ws_ref, bs_ref, wd_ref, bd_ref,
                  out_ref, *, relu_and_scale):
    inv_i = _inv_sqrt(di_ref[0])
    inv_o = _inv_sqrt(do_ref[0])
    agg_fwd = u_ref[0, 0] * inv_o
    agg_rev = u_ref[0, 1] * inv_i
    dn = (((1,), (1,)), ((), ()))
    h = (ALPHA * (lax.dot_general(agg_fwd, ws_ref[...], dn,
                                  preferred_element_type=jnp.float32)
                  + bs_ref[...])
         + (1.0 - ALPHA) * (lax.dot_general(agg_rev, wd_ref[...], dn,
                                            preferred_element_type=jnp.float32)
                            + bd_ref[...]))
    if relu_and_scale:
        h = jnp.maximum(h, 0.0)
        out_ref[0, 0] = h * inv_i
        out_ref[0, 1] = h * inv_o
    else:
        out_ref[0] = h


def _make_combine(relu_and_scale):
    if relu_and_scale:
        out_specs = pl.BlockSpec((1, 2, BN, F_DIM), lambda t, n: (t, 0, n, 0))
        out_shape = jax.ShapeDtypeStruct((T, 2, N, F_DIM), jnp.float32)
    else:
        out_specs = pl.BlockSpec((1, BN, F_DIM), lambda t, n: (t, n, 0))
        out_shape = jax.ShapeDtypeStruct((T, N, F_DIM), jnp.float32)
    return pl.pallas_call(
        functools.partial(_combine_body, relu_and_scale=relu_and_scale),
        grid=(T, NB),
        in_specs=[
            pl.BlockSpec((1, 2, BN, F_DIM), lambda t, n: (t, 0, n, 0)),
            pl.BlockSpec((1, BN, 1), lambda t, n: (t, n, 0)),
            pl.BlockSpec((1, BN, 1), lambda t, n: (t, n, 0)),
            pl.BlockSpec((H, F_DIM), lambda t, n: (0, 0)),
            pl.BlockSpec((1, H), lambda t, n: (0, 0)),
            pl.BlockSpec((H, F_DIM), lambda t, n: (0, 0)),
            pl.BlockSpec((1, H), lambda t, n: (0, 0)),
        ],
        out_specs=out_specs,
        out_shape=out_shape,
    )


_combine1_call = _make_combine(True)
_combine2_call = _make_combine(False)

BL = 2000
NBL = N // BL


def _lstm_body(seq_ref, wih_ref, whh_ref, bih_ref, bhh_ref, wp_ref, bp_ref,
               out_ref):
    dn = (((1,), (1,)), ((), ()))
    b = bih_ref[...] + bhh_ref[...]
    h = jnp.zeros((BL, H), jnp.float32)
    c = jnp.zeros((BL, H), jnp.float32)
    for t in range(T):
        xt = seq_ref[t]
        gates = (lax.dot_general(xt, wih_ref[...], dn,
                                 preferred_element_type=jnp.float32)
                 + lax.dot_general(h, whh_ref[...], dn,
                                   preferred_element_type=jnp.float32)
                 + b)
        i = jax.nn.sigmoid(gates[:, 0:H])
        f = jax.nn.sigmoid(gates[:, H:2 * H])
        g = jnp.tanh(gates[:, 2 * H:3 * H])
        o = jax.nn.sigmoid(gates[:, 3 * H:4 * H])
        c = f * c + i * g
        h = o * jnp.tanh(c)
    out_ref[...] = (lax.dot_general(h, wp_ref[...], dn,
                                    preferred_element_type=jnp.float32)
                    + bp_ref[...])


_lstm_call = pl.pallas_call(
    _lstm_body,
    grid=(NBL,),
    in_specs=[
        pl.BlockSpec((T, BL, H), lambda n: (0, n, 0)),
        pl.BlockSpec((4 * H, H), lambda n: (0, 0)),
        pl.BlockSpec((4 * H, H), lambda n: (0, 0)),
        pl.BlockSpec((1, 4 * H), lambda n: (0, 0)),
        pl.BlockSpec((1, 4 * H), lambda n: (0, 0)),
        pl.BlockSpec((F_DIM, H), lambda n: (0, 0)),
        pl.BlockSpec((1, F_DIM), lambda n: (0, 0)),
    ],
    out_specs=pl.BlockSpec((BL, F_DIM), lambda n: (n, 0)),
    out_shape=jax.ShapeDtypeStruct((N, F_DIM), jnp.float32),
)


def kernel(x_sequence, W_s1, b_s1, W_d1, b_d1, W_s2, b_s2, W_d2, b_d2,
           W_ih, W_hh, b_ih, b_hh, W_p, b_p, edge_index_sequence):
    ei = edge_index_sequence.astype(jnp.int32)
    gidx = jnp.flip(ei, axis=1)
    offs = (jnp.arange(T, dtype=jnp.int32)[:, None] * 2
            + jnp.arange(2, dtype=jnp.int32)[None, :]) * N
    gidx = gidx + offs[:, :, None]
    gidx = gidx.reshape(T, 2, NS, NCH, CHB)
    sidx = ei.reshape(T, 2, NS, NCH, CHB)

    ones_chb = jnp.ones((CHB,), jnp.float32)
    zeros_hist = jnp.zeros((N * PPC // NS,), jnp.float32)
    zeros_rows = jnp.zeros((ROWS_PT, F_DIM), jnp.float32)

    deg = _deg_call(gidx, ones_chb, zeros_hist).reshape(T, 2, N)
    deg_in = deg[:, 0, :, None]
    deg_out = deg[:, 1, :, None]

    xcat = _scale_call(x_sequence, deg_in, deg_out)
    u1 = _agg_call(xcat.reshape(PAIRS * N, F_DIM), gidx, sidx, zeros_rows)
    h1cat = _combine1_call(u1, deg_in, deg_out, W_s1, b_s1.reshape(1, H),
                           W_d1, b_d1.reshape(1, H))
    u2 = _agg_call(h1cat.reshape(PAIRS * N, F_DIM), gidx, sidx, zeros_rows)
    h2 = _combine2_call(u2, deg_in, deg_out, W_s2, b_s2.reshape(1, H),
                        W_d2, b_d2.reshape(1, H))
    out = _lstm_call(h2, W_ih, W_hh, b_ih.reshape(1, 4 * H),
                     b_hh.reshape(1, 4 * H), W_p, b_p.reshape(1, F_DIM))
    return out


# speedup vs baseline: 16.6395x; 1.2622x over previous
import functools

import jax
import jax.numpy as jnp
from jax import lax
from jax.experimental import pallas as pl
from jax.experimental.pallas import tpu as pltpu
from jax.experimental.pallas import tpu_sc as plsc

N = 10000
T = 8
F_DIM = 128
H = 128
E = 320000
ALPHA = 0.5

NC = 2
NS = 16
EPT = E // NS
CHB = 125
NCH = EPT // CHB
PAIRS = T * 2
PPC = PAIRS // NC
ROWS_PT = N // NS

_mesh = plsc.VectorSubcoreMesh(core_axis_name="c", subcore_axis_name="s")


def _deg_body(ei_hbm, ones_hbm, zeros_hbm, deg_hbm, idx_v, ones_v, bounce_v,
              sem, hist_sp):
    c = lax.axis_index("c")
    s = lax.axis_index("s")
    pltpu.sync_copy(ones_hbm, ones_v)
    pltpu.sync_copy(zeros_hbm, bounce_v)
    row = c * PPC + s // 2
    col = (s % 2) * (N // 2)
    pltpu.sync_copy(bounce_v, hist_sp.at[row, pl.ds(col, N // 2)])
    plsc.subcore_barrier()
    for tt in range(T // NC):
        for dd in range(2):
            t = c * (T // NC) + tt
            tp = t * 2 + dd
            pltpu.sync_copy(ei_hbm.at[t, 1 - dd, s], idx_v)

            @pl.loop(0, NCH)
            def _(ch):
                pltpu.sync_copy(ones_v, hist_sp.at[tp].at[idx_v.at[ch]], add=True)

    plsc.subcore_barrier()
    pltpu.sync_copy(hist_sp.at[row, pl.ds(col, N // 2)], bounce_v)
    pltpu.sync_copy(bounce_v, deg_hbm.at[row, pl.ds(col, N // 2)])


_deg_call = pl.kernel(
    _deg_body,
    out_type=jax.ShapeDtypeStruct((PAIRS, N), jnp.float32),
    mesh=_mesh,
    compiler_params=pltpu.CompilerParams(use_tc_tiling_on_sc=False),
    scratch_types=[
        pltpu.VMEM((NCH, CHB), jnp.int32),
        pltpu.VMEM((CHB,), jnp.float32),
        pltpu.VMEM((N // 2,), jnp.float32),
        pltpu.SemaphoreType.DMA,
        pltpu.VMEM_SHARED((PAIRS, N), jnp.float32),
    ],
)


NCHQ = NCH // 4


def _agg_body(table_hbm, ei_hbm, zrows_hbm, out_hbm,
              gidx_v, sidx_v, rows0_v, rows1_v, sem, acc_sp):
    c = lax.axis_index("c")
    s = lax.axis_index("s")

    def wait_rows(dst):
        pltpu.make_async_copy(table_hbm.at[0, pl.ds(0, CHB)], dst, sem).wait()

    for tt in range(T // NC):
        for dd in range(2):
            t = c * (T // NC) + tt
            tp = t * 2 + dd
            pltpu.sync_copy(zrows_hbm, acc_sp.at[pl.ds(s * ROWS_PT, ROWS_PT)])
            plsc.subcore_barrier()
            for hh in range(4):
                pltpu.sync_copy(ei_hbm.at[t, 1 - dd, s, pl.ds(hh * NCHQ, NCHQ)],
                                gidx_v)
                pltpu.sync_copy(ei_hbm.at[t, dd, s, pl.ds(hh * NCHQ, NCHQ)],
                                sidx_v)
                pltpu.async_copy(table_hbm.at[tp].at[gidx_v.at[0]], rows0_v, sem)

                @pl.loop(0, NCHQ // 2 - 1)
                def _(i):
                    ch = 2 * i
                    wait_rows(rows0_v)
                    pltpu.async_copy(table_hbm.at[tp].at[gidx_v.at[ch + 1]], rows1_v, sem)
                    pltpu.sync_copy(rows0_v, acc_sp.at[sidx_v.at[ch]], add=True)
                    wait_rows(rows1_v)
                    pltpu.async_copy(table_hbm.at[tp].at[gidx_v.at[ch + 2]], rows0_v, sem)
                    pltpu.sync_copy(rows1_v, acc_sp.at[sidx_v.at[ch + 1]], add=True)

                wait_rows(rows0_v)
                pltpu.async_copy(table_hbm.at[tp].at[gidx_v.at[NCHQ - 1]], rows1_v, sem)
                pltpu.sync_copy(rows0_v, acc_sp.at[sidx_v.at[NCHQ - 2]], add=True)
                wait_rows(rows1_v)
                pltpu.sync_copy(rows1_v, acc_sp.at[sidx_v.at[NCHQ - 1]], add=True)

            plsc.subcore_barrier()
            pltpu.sync_copy(acc_sp.at[pl.ds(s * ROWS_PT, ROWS_PT)],
                            out_hbm.at[t, dd, pl.ds(s * ROWS_PT, ROWS_PT)])


_agg_call = pl.kernel(
    _agg_body,
    out_type=jax.ShapeDtypeStruct((T, 2, N, F_DIM), jnp.float32),
    mesh=_mesh,
    compiler_params=pltpu.CompilerParams(use_tc_tiling_on_sc=False),
    scratch_types=[
        pltpu.VMEM((NCHQ, CHB), jnp.int32),
        pltpu.VMEM((NCHQ, CHB), jnp.int32),
        pltpu.VMEM((CHB, F_DIM), jnp.float32),
        pltpu.VMEM((CHB, F_DIM), jnp.float32),
        pltpu.SemaphoreType.DMA,
        pltpu.VMEM_SHARED((N, F_DIM), jnp.float32),
    ],
)


BN = 2000
NB = N // BN


def _inv_sqrt(d):
    return jnp.where(d > 0, lax.rsqrt(jnp.maximum(d, 1e-12)), 0.0)


def _scale_body(x_ref, di_ref, do_ref, out_ref):
    x = x_ref[0]
    inv_i = _inv_sqrt(di_ref[0])
    inv_o = _inv_sqrt(do_ref[0])
    out_ref[0, 0] = x * inv_i
    out_ref[0, 1] = x * inv_o


_scale_call = pl.pallas_call(
    _scale_body,
    grid=(T, NB),
    in_specs=[
        pl.BlockSpec((1, BN, F_DIM), lambda t, n: (t, n, 0)),
        pl.BlockSpec((1, BN, 1), lambda t, n: (t, n, 0)),
        pl.BlockSpec((1, BN, 1), lambda t, n: (t, n, 0)),
    ],
    out_specs=pl.BlockSpec((1, 2, BN, F_DIM), lambda t, n: (t, 0, n, 0)),
    out_shape=jax.ShapeDtypeStruct((T, 2, N, F_DIM), jnp.float32),
)


def _combine_body(u_ref, di_ref, do_ref, ws_ref, bs_ref, wd_ref, bd_ref,
                  out_ref, *, relu_and_scale):
    inv_i = _inv_sqrt(di_ref[0])
    inv_o = _inv_sqrt(do_ref[0])
    agg_fwd = u_ref[0, 0] * inv_o
    agg_rev = u_ref[0, 1] * inv_i
    dn = (((1,), (1,)), ((), ()))
    h = (ALPHA * (lax.dot_general(agg_fwd, ws_ref[...], dn,
                                  preferred_element_type=jnp.float32)
                  + bs_ref[...])
         + (1.0 - ALPHA) * (lax.dot_general(agg_rev, wd_ref[...], dn,
                                            preferred_element_type=jnp.float32)
                            + bd_ref[...]))
    if relu_and_scale:
        h = jnp.maximum(h, 0.0)
        out_ref[0, 0] = h * inv_i
        out_ref[0, 1] = h * inv_o
    else:
        out_ref[0] = h


def _make_combine(relu_and_scale):
    if relu_and_scale:
        out_specs = pl.BlockSpec((1, 2, BN, F_DIM), lambda t, n: (t, 0, n, 0))
        out_shape = jax.ShapeDtypeStruct((T, 2, N, F_DIM), jnp.float32)
    else:
        out_specs = pl.BlockSpec((1, BN, F_DIM), lambda t, n: (t, n, 0))
        out_shape = jax.ShapeDtypeStruct((T, N, F_DIM), jnp.float32)
    return pl.pallas_call(
        functools.partial(_combine_body, relu_and_scale=relu_and_scale),
        grid=(T, NB),
        in_specs=[
            pl.BlockSpec((1, 2, BN, F_DIM), lambda t, n: (t, 0, n, 0)),
            pl.BlockSpec((1, BN, 1), lambda t, n: (t, n, 0)),
            pl.BlockSpec((1, BN, 1), lambda t, n: (t, n, 0)),
            pl.BlockSpec((H, F_DIM), lambda t, n: (0, 0)),
            pl.BlockSpec((1, H), lambda t, n: (0, 0)),
            pl.BlockSpec((H, F_DIM), lambda t, n: (0, 0)),
            pl.BlockSpec((1, H), lambda t, n: (0, 0)),
        ],
        out_specs=out_specs,
        out_shape=out_shape,
    )


_combine1_call = _make_combine(True)
_combine2_call = _make_combine(False)

BL = 2000
NBL = N // BL


def _lstm_body(seq_ref, wih_ref, whh_ref, bih_ref, bhh_ref, wp_ref, bp_ref,
               out_ref):
    dn = (((1,), (1,)), ((), ()))
    b = bih_ref[...] + bhh_ref[...]
    h = jnp.zeros((BL, H), jnp.float32)
    c = jnp.zeros((BL, H), jnp.float32)
    for t in range(T):
        xt = seq_ref[t]
        gates = (lax.dot_general(xt, wih_ref[...], dn,
                                 preferred_element_type=jnp.float32)
                 + lax.dot_general(h, whh_ref[...], dn,
                                   preferred_element_type=jnp.float32)
                 + b)
        i = jax.nn.sigmoid(gates[:, 0:H])
        f = jax.nn.sigmoid(gates[:, H:2 * H])
        g = jnp.tanh(gates[:, 2 * H:3 * H])
        o = jax.nn.sigmoid(gates[:, 3 * H:4 * H])
        c = f * c + i * g
        h = o * jnp.tanh(c)
    out_ref[...] = (lax.dot_general(h, wp_ref[...], dn,
                                    preferred_element_type=jnp.float32)
                    + bp_ref[...])


_lstm_call = pl.pallas_call(
    _lstm_body,
    grid=(NBL,),
    in_specs=[
        pl.BlockSpec((T, BL, H), lambda n: (0, n, 0)),
        pl.BlockSpec((4 * H, H), lambda n: (0, 0)),
        pl.BlockSpec((4 * H, H), lambda n: (0, 0)),
        pl.BlockSpec((1, 4 * H), lambda n: (0, 0)),
        pl.BlockSpec((1, 4 * H), lambda n: (0, 0)),
        pl.BlockSpec((F_DIM, H), lambda n: (0, 0)),
        pl.BlockSpec((1, F_DIM), lambda n: (0, 0)),
    ],
    out_specs=pl.BlockSpec((BL, F_DIM), lambda n: (n, 0)),
    out_shape=jax.ShapeDtypeStruct((N, F_DIM), jnp.float32),
)


def kernel(x_sequence, W_s1, b_s1, W_d1, b_d1, W_s2, b_s2, W_d2, b_d2,
           W_ih, W_hh, b_ih, b_hh, W_p, b_p, edge_index_sequence):
    ei5 = edge_index_sequence.reshape(T, 2, NS, NCH, CHB)

    ones_chb = jnp.ones((CHB,), jnp.float32)
    zeros_hist = jnp.zeros((N // 2,), jnp.float32)
    zeros_rows = jnp.zeros((ROWS_PT, F_DIM), jnp.float32)

    deg = _deg_call(ei5, ones_chb, zeros_hist).reshape(T, 2, N)
    deg_in = deg[:, 0, :, None]
    deg_out = deg[:, 1, :, None]

    xcat = _scale_call(x_sequence, deg_in, deg_out)
    u1 = _agg_call(xcat.reshape(PAIRS, N, F_DIM), ei5, zeros_rows)
    h1cat = _combine1_call(u1, deg_in, deg_out, W_s1, b_s1.reshape(1, H),
                           W_d1, b_d1.reshape(1, H))
    u2 = _agg_call(h1cat.reshape(PAIRS, N, F_DIM), ei5, zeros_rows)
    h2 = _combine2_call(u2, deg_in, deg_out, W_s2, b_s2.reshape(1, H),
                        W_d2, b_d2.reshape(1, H))
    out = _lstm_call(h2, W_ih, W_hh, b_ih.reshape(1, 4 * H),
                     b_hh.reshape(1, 4 * H), W_p, b_p.reshape(1, F_DIM))
    return out


# speedup vs baseline: 19.7140x; 1.1848x over previous
import functools

import jax
import jax.numpy as jnp
from jax import lax
from jax.experimental import pallas as pl
from jax.experimental.pallas import tpu as pltpu
from jax.experimental.pallas import tpu_sc as plsc

N = 10000
T = 8
F_DIM = 128
H = 128
E = 320000
ALPHA = 0.5

NC = 2
NS = 16
EPT = E // NS
CHB = 125
NCH = EPT // CHB
PAIRS = T * 2
PPC = PAIRS // NC
ROWS_PT = N // NS

_mesh = plsc.VectorSubcoreMesh(core_axis_name="c", subcore_axis_name="s")


def _deg_body(ei_hbm, ones_hbm, zeros_hbm, deg_hbm, idx_v, ones_v, bounce_v,
              sem, hist_sp):
    c = lax.axis_index("c")
    s = lax.axis_index("s")
    pltpu.sync_copy(ones_hbm, ones_v)
    pltpu.sync_copy(zeros_hbm, bounce_v)
    row = c * PPC + s // 2
    col = (s % 2) * (N // 2)
    pltpu.sync_copy(bounce_v, hist_sp.at[row, pl.ds(col, N // 2)])
    plsc.subcore_barrier()
    for tt in range(T // NC):
        for dd in range(2):
            t = c * (T // NC) + tt
            tp = t * 2 + dd
            pltpu.sync_copy(ei_hbm.at[t, 1 - dd, s], idx_v)

            @pl.loop(0, NCH)
            def _(ch):
                pltpu.sync_copy(ones_v, hist_sp.at[tp].at[idx_v.at[ch]], add=True)

    plsc.subcore_barrier()
    pltpu.sync_copy(hist_sp.at[row, pl.ds(col, N // 2)], bounce_v)
    pltpu.sync_copy(bounce_v, deg_hbm.at[row, pl.ds(col, N // 2)])


_deg_call = pl.kernel(
    _deg_body,
    out_type=jax.ShapeDtypeStruct((PAIRS, N), jnp.float32),
    mesh=_mesh,
    compiler_params=pltpu.CompilerParams(use_tc_tiling_on_sc=False),
    scratch_types=[
        pltpu.VMEM((NCH, CHB), jnp.int32),
        pltpu.VMEM((CHB,), jnp.float32),
        pltpu.VMEM((N // 2,), jnp.float32),
        pltpu.SemaphoreType.DMA,
        pltpu.VMEM_SHARED((PAIRS, N), jnp.float32),
    ],
)


NCHQ = NCH // 4


def _agg_body(table_hbm, ei_hbm, zrows_hbm, out_hbm,
              gidx_v, sidx_v, rows_a, rows_b, sem_ga, sem_gb, sem_sa, sem_sb,
              acc_sp):
    c = lax.axis_index("c")
    s = lax.axis_index("s")

    def g(ch, buf, sem):
        pltpu.async_copy(table_hbm.at[0].at[gidx_v.at[ch]], buf, sem)

    def wait_g(buf, sem):
        pltpu.make_async_copy(table_hbm.at[0, pl.ds(0, CHB)], buf, sem).wait()

    def sc(ch, buf, sem):
        pltpu.async_copy(buf, acc_sp.at[sidx_v.at[ch]], sem, add=True)

    def wait_s(buf, sem):
        pltpu.make_async_copy(buf, acc_sp.at[sidx_v.at[0]], sem).wait()

    for tt in range(T // NC):
        for dd in range(2):
            t = c * (T // NC) + tt
            tp = t * 2 + dd
            tbl = table_hbm.at[tp]
            pltpu.sync_copy(zrows_hbm, acc_sp.at[pl.ds(s * ROWS_PT, ROWS_PT)])
            plsc.subcore_barrier()
            for hh in range(4):
                pltpu.sync_copy(ei_hbm.at[t, 1 - dd, s, pl.ds(hh * NCHQ, NCHQ)],
                                gidx_v)
                pltpu.sync_copy(ei_hbm.at[t, dd, s, pl.ds(hh * NCHQ, NCHQ)],
                                sidx_v)

                def gg(ch, buf, sem):
                    pltpu.async_copy(tbl.at[gidx_v.at[ch]], buf, sem)

                gg(0, rows_a, sem_ga)
                gg(1, rows_b, sem_gb)
                wait_g(rows_a, sem_ga)
                sc(0, rows_a, sem_sa)

                @pl.loop(0, NCHQ // 2 - 1)
                def _(i):
                    ch = 2 * i
                    wait_s(rows_a, sem_sa)
                    gg(ch + 2, rows_a, sem_ga)
                    wait_g(rows_b, sem_gb)
                    sc(ch + 1, rows_b, sem_sb)
                    wait_s(rows_b, sem_sb)
                    gg(ch + 3, rows_b, sem_gb)
                    wait_g(rows_a, sem_ga)
                    sc(ch + 2, rows_a, sem_sa)

                wait_s(rows_a, sem_sa)
                wait_g(rows_b, sem_gb)
                sc(NCHQ - 1, rows_b, sem_sb)
                wait_s(rows_b, sem_sb)

            plsc.subcore_barrier()
            pltpu.sync_copy(acc_sp.at[pl.ds(s * ROWS_PT, ROWS_PT)],
                            out_hbm.at[t, dd, pl.ds(s * ROWS_PT, ROWS_PT)])


_agg_call = pl.kernel(
    _agg_body,
    out_type=jax.ShapeDtypeStruct((T, 2, N, F_DIM), jnp.float32),
    mesh=_mesh,
    compiler_params=pltpu.CompilerParams(use_tc_tiling_on_sc=False),
    scratch_types=[
        pltpu.VMEM((NCHQ, CHB), jnp.int32),
        pltpu.VMEM((NCHQ, CHB), jnp.int32),
        pltpu.VMEM((CHB, F_DIM), jnp.float32),
        pltpu.VMEM((CHB, F_DIM), jnp.float32),
        pltpu.SemaphoreType.DMA,
        pltpu.SemaphoreType.DMA,
        pltpu.SemaphoreType.DMA,
        pltpu.SemaphoreType.DMA,
        pltpu.VMEM_SHARED((N, F_DIM), jnp.float32),
    ],
)


BN = 2000
NB = N // BN


def _inv_sqrt(d):
    return jnp.where(d > 0, lax.rsqrt(jnp.maximum(d, 1e-12)), 0.0)


def _scale_body(x_ref, di_ref, do_ref, out_ref):
    x = x_ref[0]
    inv_i = _inv_sqrt(di_ref[0])
    inv_o = _inv_sqrt(do_ref[0])
    out_ref[0, 0] = x * inv_i
    out_ref[0, 1] = x * inv_o


_scale_call = pl.pallas_call(
    _scale_body,
    grid=(T, NB),
    in_specs=[
        pl.BlockSpec((1, BN, F_DIM), lambda t, n: (t, n, 0)),
        pl.BlockSpec((1, BN, 1), lambda t, n: (t, n, 0)),
        pl.BlockSpec((1, BN, 1), lambda t, n: (t, n, 0)),
    ],
    out_specs=pl.BlockSpec((1, 2, BN, F_DIM), lambda t, n: (t, 0, n, 0)),
    out_shape=jax.ShapeDtypeStruct((T, 2, N, F_DIM), jnp.float32),
)


def _combine_body(u_ref, di_ref, do_ref, ws_ref, bs_ref, wd_ref, bd_ref,
                  out_ref, *, relu_and_scale):
    inv_i = _inv_sqrt(di_ref[0])
    inv_o = _inv_sqrt(do_ref[0])
    agg_fwd = u_ref[0, 0] * inv_o
    agg_rev = u_ref[0, 1] * inv_i
    dn = (((1,), (1,)), ((), ()))
    h = (ALPHA * (lax.dot_general(agg_fwd, ws_ref[...], dn,
                                  preferred_element_type=jnp.float32)
                  + bs_ref[...])
         + (1.0 - ALPHA) * (lax.dot_general(agg_rev, wd_ref[...], dn,
                                            preferred_element_type=jnp.float32)
                            + bd_ref[...]))
    if relu_and_scale:
        h = jnp.maximum(h, 0.0)
        out_ref[0, 0] = h * inv_i
        out_ref[0, 1] = h * inv_o
    else:
        out_ref[0] = h


def _make_combine(relu_and_scale):
    if relu_and_scale:
        out_specs = pl.BlockSpec((1, 2, BN, F_DIM), lambda t, n: (t, 0, n, 0))
        out_shape = jax.ShapeDtypeStruct((T, 2, N, F_DIM), jnp.float32)
    else:
        out_specs = pl.BlockSpec((1, BN, F_DIM), lambda t, n: (t, n, 0))
        out_shape = jax.ShapeDtypeStruct((T, N, F_DIM), jnp.float32)
    return pl.pallas_call(
        functools.partial(_combine_body, relu_and_scale=relu_and_scale),
        grid=(T, NB),
        in_specs=[
            pl.BlockSpec((1, 2, BN, F_DIM), lambda t, n: (t, 0, n, 0)),
            pl.BlockSpec((1, BN, 1), lambda t, n: (t, n, 0)),
            pl.BlockSpec((1, BN, 1), lambda t, n: (t, n, 0)),
            pl.BlockSpec((H, F_DIM), lambda t, n: (0, 0)),
            pl.BlockSpec((1, H), lambda t, n: (0, 0)),
            pl.BlockSpec((H, F_DIM), lambda t, n: (0, 0)),
            pl.BlockSpec((1, H), lambda t, n: (0, 0)),
        ],
        out_specs=out_specs,
        out_shape=out_shape,
    )


_combine1_call = _make_combine(True)
_combine2_call = _make_combine(False)

BL = 2000
NBL = N // BL


def _lstm_body(seq_ref, wih_ref, whh_ref, bih_ref, bhh_ref, wp_ref, bp_ref,
               out_ref):
    dn = (((1,), (1,)), ((), ()))
    b = bih_ref[...] + bhh_ref[...]
    h = jnp.zeros((BL, H), jnp.float32)
    c = jnp.zeros((BL, H), jnp.float32)
    for t in range(T):
        xt = seq_ref[t]
        gates = (lax.dot_general(xt, wih_ref[...], dn,
                                 preferred_element_type=jnp.float32)
                 + lax.dot_general(h, whh_ref[...], dn,
                                   preferred_element_type=jnp.float32)
                 + b)
        i = jax.nn.sigmoid(gates[:, 0:H])
        f = jax.nn.sigmoid(gates[:, H:2 * H])
        g = jnp.tanh(gates[:, 2 * H:3 * H])
        o = jax.nn.sigmoid(gates[:, 3 * H:4 * H])
        c = f * c + i * g
        h = o * jnp.tanh(c)
    out_ref[...] = (lax.dot_general(h, wp_ref[...], dn,
                                    preferred_element_type=jnp.float32)
                    + bp_ref[...])


_lstm_call = pl.pallas_call(
    _lstm_body,
    grid=(NBL,),
    in_specs=[
        pl.BlockSpec((T, BL, H), lambda n: (0, n, 0)),
        pl.BlockSpec((4 * H, H), lambda n: (0, 0)),
        pl.BlockSpec((4 * H, H), lambda n: (0, 0)),
        pl.BlockSpec((1, 4 * H), lambda n: (0, 0)),
        pl.BlockSpec((1, 4 * H), lambda n: (0, 0)),
        pl.BlockSpec((F_DIM, H), lambda n: (0, 0)),
        pl.BlockSpec((1, F_DIM), lambda n: (0, 0)),
    ],
    out_specs=pl.BlockSpec((BL, F_DIM), lambda n: (n, 0)),
    out_shape=jax.ShapeDtypeStruct((N, F_DIM), jnp.float32),
)


def kernel(x_sequence, W_s1, b_s1, W_d1, b_d1, W_s2, b_s2, W_d2, b_d2,
           W_ih, W_hh, b_ih, b_hh, W_p, b_p, edge_index_sequence):
    ei5 = edge_index_sequence.reshape(T, 2, NS, NCH, CHB)

    ones_chb = jnp.ones((CHB,), jnp.float32)
    zeros_hist = jnp.zeros((N // 2,), jnp.float32)
    zeros_rows = jnp.zeros((ROWS_PT, F_DIM), jnp.float32)

    deg = _deg_call(ei5, ones_chb, zeros_hist).reshape(T, 2, N)
    deg_in = deg[:, 0, :, None]
    deg_out = deg[:, 1, :, None]

    xcat = _scale_call(x_sequence, deg_in, deg_out)
    u1 = _agg_call(xcat.reshape(PAIRS, N, F_DIM), ei5, zeros_rows)
    h1cat = _combine1_call(u1, deg_in, deg_out, W_s1, b_s1.reshape(1, H),
                           W_d1, b_d1.reshape(1, H))
    u2 = _agg_call(h1cat.reshape(PAIRS, N, F_DIM), ei5, zeros_rows)
    h2 = _combine2_call(u2, deg_in, deg_out, W_s2, b_s2.reshape(1, H),
                        W_d2, b_d2.reshape(1, H))
    out = _lstm_call(h2, W_ih, W_hh, b_ih.reshape(1, 4 * H),
                     b_hh.reshape(1, 4 * H), W_p, b_p.reshape(1, F_DIM))
    return out
